# baseline (speedup 1.0000x reference)
import functools

import jax
import jax.numpy as jnp
from jax import lax
from jax.experimental import pallas as pl
from jax.experimental.pallas import tpu as pltpu
from jax.experimental.pallas import tpu_sc as plsc

_B = 128
_N = 32768
_L = 16
_NSLICES = _N // _L
_NWORKERS = 32
_RPW = _B // _NWORKERS
_CAPL = 512
_CAP2 = 128
_ZCH = 8192
_PADB = _B * _N

_mesh = plsc.VectorSubcoreMesh(core_axis_name="c", subcore_axis_name="s")


@functools.partial(
    pl.kernel,
    out_type=jax.ShapeDtypeStruct((_B * _N + _L,), jnp.float32),
    mesh=_mesh,
    scratch_types=[
        pltpu.VMEM((_N,), jnp.float32),
        pltpu.VMEM((_N,), jnp.float32),
        pltpu.VMEM((_ZCH,), jnp.float32),
        pltpu.VMEM((_CAPL * _L,), jnp.int32),
        pltpu.VMEM((_CAP2 * _L,), jnp.int32),
        pltpu.VMEM((_L, 128), jnp.int32),
        pltpu.VMEM((_L, 128), jnp.int32),
        pltpu.VMEM((_L, 128), jnp.float32),
        pltpu.VMEM((_L, 128), jnp.float32),
        pltpu.SemaphoreType.DMA,
        pltpu.SemaphoreType.DMA,
        pltpu.SemaphoreType.DMA,
        pltpu.SemaphoreType.DMA,
        pltpu.SemaphoreType.DMA,
        pltpu.SemaphoreType.DMA,
        pltpu.SemaphoreType.DMA,
        pltpu.SemaphoreType.DMA,
        pltpu.SemaphoreType.DMA,
    ],
    compiler_params=pltpu.CompilerParams(needs_layout_passes=False),
)
def _sparsemax_sc(x_hbm, out_hbm, row0_v, row1_v, zero_v, cidx_v, c2idx_v,
                  sidx0_v, sidx1_v, sval0_v, sval1_v,
                  isem0, isem1, zsem0, zsem1, zsem2, zsem3,
                  fsem0, fsem1, fsem2):
    wid = lax.axis_index("s") * 2 + lax.axis_index("c")
    iota = lax.broadcasted_iota(jnp.int32, (_L,), 0)
    zf = jnp.zeros((_L,), jnp.float32)
    wrap1 = jnp.int32(_CAPL * _L - 1)
    wrap2 = jnp.int32(_CAP2 * _L - 1)
    pad_idx = _PADB + iota

    bufs = [row0_v, row1_v]
    isems = [isem0, isem1]
    zsems = [zsem0, zsem1, zsem2, zsem3]
    sidxs = [sidx0_v, sidx1_v]
    svals = [sval0_v, sval1_v]
    fsems = [fsem0, fsem1]
    base = wid * _RPW

    in_h = {r: pltpu.async_copy(x_hbm.at[base + r], bufs[r], isems[r])
            for r in range(2)}

    def _zfill(i):
        zero_v[pl.ds(i * _L, _L)] = zf

    plsc.parallel_loop(0, _ZCH // _L, unroll=8)(_zfill)

    def _sfill(i):
        q, c = i // 8, (i % 8) * _L
        for sv in sidxs:
            sv[q, pl.ds(c, _L)] = pad_idx
        for sv in svals:
            sv[q, pl.ds(c, _L)] = zf

    plsc.parallel_loop(0, 128, unroll=4)(_sfill)

    z_h = {}
    for r in range(_RPW):
        z_h[r] = [pltpu.async_copy(
            zero_v, out_hbm.at[pl.ds((base + r) * _N + k * _ZCH, _ZCH)],
            zsems[r]) for k in range(_N // _ZCH)]

    f_h = {}
    for r in range(_RPW):
        row = base + r
        row_v = bufs[r % 2]
        par = r % 2
        sidx_v, sval_v = sidxs[par], svals[par]
        in_h[r].wait()

        def _fuse(i, carry):
            vmax_m1, pos = carry
            v = row_v[pl.ds(i * _L, _L)]
            msk = v > vmax_m1
            vmax_m1 = jnp.maximum(vmax_m1, v - 1.0)
            plsc.store_scatter(cidx_v, [pos & wrap1], i * _L + iota, mask=msk)
            pos = pos + jnp.where(msk, _L, 0)
            return (vmax_m1, pos)

        vmax_m1, pos = plsc.parallel_loop(
            0, _NSLICES, unroll=8,
            carry=(jnp.full((_L,), -3.0e38, jnp.float32), iota))(_fuse)
        lane_cnt = (pos - iota) // _L
        maxcnt = jnp.max(lane_cnt)
        ok1 = maxcnt <= _CAPL
        m = jnp.max(vmax_m1) + 1.0
        t0 = m - 1.0

        def _fk0(j, carry):
            s_acc, k_acc, pos2 = carry
            valid = j < lane_cnt
            idx = cidx_v[pl.ds(j * _L, _L)]
            v = plsc.load_gather(row_v, [jnp.where(valid, idx, 0)])
            msk = (v > t0) & valid
            plsc.store_scatter(c2idx_v, [pos2 & wrap2], idx, mask=msk)
            pos2 = pos2 + jnp.where(msk, _L, 0)
            return (s_acc + jnp.where(msk, v, 0.0),
                    k_acc + jnp.where(msk, 1.0, 0.0), pos2)

        bound0 = jnp.where(ok1, maxcnt, 0)
        s_acc, k_acc, pos2 = lax.fori_loop(0, bound0, _fk0, (zf, zf, iota))
        lane_cnt2 = (pos2 - iota) // _L
        maxcnt2 = jnp.max(lane_cnt2)
        ok = ok1 & (maxcnt2 <= _CAP2)
        s0, k0 = jnp.sum(s_acc), jnp.sum(k_acc)
        t1 = jnp.maximum(
            t0, ((jnp.full((_L,), s0) - 1.0) / jnp.full((_L,), k0))[0])

        def michelot(fk, t_from, t_prev0):
            def cond(carry):
                t_prev, t = carry
                return t > t_prev

            def body(carry):
                _, t = carry
                s, k = fk(t)
                t_new = ((jnp.full((_L,), s) - 1.0) / jnp.full((_L,), k))[0]
                return (t, jnp.maximum(t, t_new))

            return lax.while_loop(cond, body, (t_prev0, t_from))[1]

        def fast_tau(_):
            def fk(t):
                def b(j, acc):
                    s_acc, k_acc = acc
                    valid = j < lane_cnt2
                    idx = c2idx_v[pl.ds(j * _L, _L)]
                    v = plsc.load_gather(row_v, [jnp.where(valid, idx, 0)])
                    msk = (v > t) & valid
                    return (s_acc + jnp.where(msk, v, 0.0),
                            k_acc + jnp.where(msk, 1.0, 0.0))

                s_acc, k_acc = lax.fori_loop(0, maxcnt2, b, (zf, zf))
                return jnp.sum(s_acc), jnp.sum(k_acc)

            return michelot(fk, t1, t0)

        def slow_tau(_):
            def fk(t):
                def b(j, acc):
                    s_acc, k_acc = acc
                    v = row_v[pl.ds(j * _L, _L)]
                    msk = v > t
                    return (s_acc + jnp.where(msk, v, 0.0),
                            k_acc + jnp.where(msk, 1.0, 0.0))

                s_acc, k_acc = lax.fori_loop(0, _NSLICES, b, (zf, zf))
                return jnp.sum(s_acc), jnp.sum(k_acc)

            return michelot(fk, t0, t0 - 1.0)

        tau = lax.cond(ok, fast_tau, slow_tau, 0)

        if r - 2 in f_h:
            f_h[r - 2].wait()

        @pl.when(ok)
        def _prep():
            def b(j, _):
                valid = j < lane_cnt2
                idx = c2idx_v[pl.ds(j * _L, _L)]
                v = plsc.load_gather(row_v, [jnp.where(valid, idx, 0)])
                o = jnp.maximum(v - tau, 0.0)
                gidx = jnp.where(valid, row * _N + idx, pad_idx)
                q, c = j // 8, (j % 8) * _L
                sidx_v[q, pl.ds(c, _L)] = gidx
                sval_v[q, pl.ds(c, _L)] = jnp.where(valid, o, 0.0)
                return 0

            lax.fori_loop(0, maxcnt2, b, 0)

        for h in z_h[r]:
            h.wait()

        f_h[r] = pltpu.async_copy(
            sval_v.at[0], out_hbm.at[sidx_v.at[0]], fsems[par])
        nchunks = jnp.where(ok, (maxcnt2 * _L + 127) // 128, 1)

        def _extra(c, _):
            pltpu.async_copy(
                sval_v.at[c], out_hbm.at[sidx_v.at[c]], fsem2).wait()
            return 0

        lax.fori_loop(1, nchunks, _extra, 0)

        @pl.when(jnp.logical_not(ok))
        def _dense():
            def b(j, _):
                v = row_v[pl.ds(j * _L, _L)]
                row_v[pl.ds(j * _L, _L)] = jnp.maximum(v - tau, 0.0)
                return 0

            lax.fori_loop(0, _NSLICES, b, 0)
            pltpu.sync_copy(row_v, out_hbm.at[pl.ds(row * _N, _N)])

        if r + 2 < _RPW:
            in_h[r + 2] = pltpu.async_copy(
                x_hbm.at[base + r + 2], bufs[r % 2], isems[r % 2])

    f_h[_RPW - 2].wait()
    f_h[_RPW - 1].wait()


def kernel(input):
    flat = _sparsemax_sc(input)
    return flat[:_B * _N].reshape(_B, _N)


# speedup vs baseline: 44.8787x; 44.8787x over previous
import functools

import jax
import jax.numpy as jnp
from jax import lax
from jax.experimental import pallas as pl
from jax.experimental.pallas import tpu as pltpu
from jax.experimental.pallas import tpu_sc as plsc

_B = 128
_N = 32768
_L = 16
_NSLICES = _N // _L
_NWORKERS = 32
_ROWS_PER_WORKER = _B // _NWORKERS
_CAPL = 1024
_CAP2 = 128

_mesh = plsc.VectorSubcoreMesh(core_axis_name="c", subcore_axis_name="s")


@functools.partial(
    pl.kernel,
    out_type=jax.ShapeDtypeStruct((_B, _N), jnp.float32),
    mesh=_mesh,
    scratch_types=[
        pltpu.VMEM((_N,), jnp.float32),
        pltpu.VMEM((_N,), jnp.float32),
        pltpu.VMEM((_N,), jnp.float32),
        pltpu.VMEM((_CAPL * _L,), jnp.float32),
        pltpu.VMEM((_CAP2 * _L,), jnp.float32),
        pltpu.SemaphoreType.DMA,
        pltpu.SemaphoreType.DMA,
        pltpu.SemaphoreType.DMA,
        pltpu.SemaphoreType.DMA,
        pltpu.SemaphoreType.DMA,
        pltpu.SemaphoreType.DMA,
    ],
    compiler_params=pltpu.CompilerParams(needs_layout_passes=False),
)
def _sparsemax_sc(x_hbm, out_hbm, row0_v, row1_v, row2_v, cval_v, c2val_v,
                  isem0, isem1, isem2, osem0, osem1, osem2):
    wid = lax.axis_index("s") * 2 + lax.axis_index("c")
    iota = lax.broadcasted_iota(jnp.int32, (_L,), 0)
    zf = jnp.zeros((_L,), jnp.float32)
    wrap = jnp.int32(_CAPL * _L - 1)
    wrap2 = jnp.int32(_CAP2 * _L - 1)

    bufs = [row0_v, row1_v, row2_v]
    isems = [isem0, isem1, isem2]
    osems = [osem0, osem1, osem2]
    in_h = {r: pltpu.async_copy(x_hbm.at[wid * _ROWS_PER_WORKER + r],
                                bufs[r], isems[r])
            for r in range(3)}
    out_h = {}

    for r in range(_ROWS_PER_WORKER):
        row = wid * _ROWS_PER_WORKER + r
        row_v = bufs[r % 3]
        in_h[r].wait()

        def _fuse(i, carry):
            vmax_m1, pos = carry
            v = row_v[pl.ds(i * _L, _L)]
            msk = v > vmax_m1
            vmax_m1 = jnp.maximum(vmax_m1, v - 1.0)
            plsc.store_scatter(cval_v, [pos & wrap], v, mask=msk)
            pos = pos + jnp.where(msk, _L, 0)
            return (vmax_m1, pos)

        vmax_m1, pos = plsc.parallel_loop(
            0, _NSLICES, unroll=8,
            carry=(jnp.full((_L,), -3.0e38, jnp.float32), iota))(_fuse)

        nxt = r + 2
        if 3 <= nxt < _ROWS_PER_WORKER:
            out_h[nxt - 3].wait()
            in_h[nxt] = pltpu.async_copy(
                x_hbm.at[wid * _ROWS_PER_WORKER + nxt], bufs[nxt % 3],
                isems[nxt % 3])
        lane_cnt = (pos - iota) // _L
        maxcnt = jnp.max(lane_cnt)
        ok1 = maxcnt <= _CAPL
        m = jnp.max(vmax_m1) + 1.0
        t0 = m - 1.0

        def _fk0(j, carry):
            s_acc, k_acc, pos2 = carry
            v = cval_v[pl.ds(j * _L, _L)]
            msk = (v > t0) & (j < lane_cnt)
            plsc.store_scatter(c2val_v, [pos2 & wrap2], v, mask=msk)
            pos2 = pos2 + jnp.where(msk, _L, 0)
            return (s_acc + jnp.where(msk, v, 0.0),
                    k_acc + jnp.where(msk, 1.0, 0.0), pos2)

        bound0 = jnp.where(ok1, maxcnt, 0)
        s_acc, k_acc, pos2 = lax.fori_loop(0, bound0, _fk0, (zf, zf, iota))
        lane_cnt2 = (pos2 - iota) // _L
        maxcnt2 = jnp.max(lane_cnt2)
        ok = ok1 & (maxcnt2 <= _CAP2)
        s0, k0 = jnp.sum(s_acc), jnp.sum(k_acc)
        t1 = jnp.maximum(
            t0, ((jnp.full((_L,), s0) - 1.0) / jnp.full((_L,), k0))[0])

        def michelot(fk, t_from, t_prev0):
            def cond(carry):
                t_prev, t = carry
                return t > t_prev

            def body(carry):
                _, t = carry
                s, k = fk(t)
                t_new = ((jnp.full((_L,), s) - 1.0) / jnp.full((_L,), k))[0]
                return (t, jnp.maximum(t, t_new))

            return lax.while_loop(cond, body, (t_prev0, t_from))[1]

        def fast_tau(_):
            def fk(t):
                def b(j, acc):
                    s_acc, k_acc = acc
                    v = c2val_v[pl.ds(j * _L, _L)]
                    msk = (v > t) & (j < lane_cnt2)
                    return (s_acc + jnp.where(msk, v, 0.0),
                            k_acc + jnp.where(msk, 1.0, 0.0))

                s_acc, k_acc = lax.fori_loop(0, maxcnt2, b, (zf, zf))
                return jnp.sum(s_acc), jnp.sum(k_acc)

            return michelot(fk, t1, t0)

        def slow_tau(_):
            def fk(t):
                def b(j, acc):
                    s_acc, k_acc = acc
                    v = row_v[pl.ds(j * _L, _L)]
                    msk = v > t
                    return (s_acc + jnp.where(msk, v, 0.0),
                            k_acc + jnp.where(msk, 1.0, 0.0))

                s_acc, k_acc = lax.fori_loop(0, _NSLICES, b, (zf, zf))
                return jnp.sum(s_acc), jnp.sum(k_acc)

            return michelot(fk, t0, t0 - 1.0)

        tau = lax.cond(ok, fast_tau, slow_tau, 0)

        def _out(i):
            v = row_v[pl.ds(i * _L, _L)]
            row_v[pl.ds(i * _L, _L)] = jnp.maximum(v - tau, 0.0)

        plsc.parallel_loop(0, _NSLICES, unroll=8)(_out)

        out_h[r] = pltpu.async_copy(row_v, out_hbm.at[row], osems[r % 3])

    for r in range(max(0, _ROWS_PER_WORKER - 3), _ROWS_PER_WORKER):
        out_h[r].wait()


def kernel(input):
    return _sparsemax_sc(input)
